# baseline (speedup 1.0000x reference)
import functools

import jax
import jax.numpy as jnp
from jax import lax
from jax.experimental import pallas as pl
from jax.experimental.pallas import tpu as pltpu
from jax.experimental.pallas import tpu_sc as plsc

N = 10000
E = 320000
F = 128
FH = F // 2

C = 128
TILES = 16
NCHUNK = 160
NPAIR = NCHUNK // 2
EPAD = TILES * NCHUNK * C
OUTN = 10240
NACC = OUTN
ZROWS = NACC // TILES
ORS = OUTN // TILES

BR = 1000
NB = N // BR


def _build_tables_body(h_ref, fd_ref, wr_ref, ur_ref, br_ref, ta_ref, tb_ref):
    i = pl.program_id(0)

    @pl.when(i < NB)
    def _copy():
        hv = h_ref[...]
        ta_ref[...] = hv[:, :FH]
        tb_ref[...] = hv[:, FH:]

    @pl.when(i >= NB)
    def _compute():
        hv = h_ref[...]
        r = jax.nn.sigmoid(
            jnp.dot(fd_ref[...], wr_ref[...], preferred_element_type=jnp.float32)
            + jnp.dot(hv, ur_ref[...], preferred_element_type=jnp.float32)
            + br_ref[...]
        )
        rh = r * hv
        ta_ref[...] = rh[:, :FH]
        tb_ref[...] = rh[:, FH:]


def _build_tables(h, f_dst, wr, ur, br):
    return pl.pallas_call(
        _build_tables_body,
        grid=(2 * NB,),
        in_specs=[
            pl.BlockSpec((BR, F), lambda i: (jnp.where(i < NB, i, i - NB), 0)),
            pl.BlockSpec((BR, F), lambda i: (jnp.where(i < NB, 0, i - NB), 0)),
            pl.BlockSpec((F, F), lambda i: (0, 0)),
            pl.BlockSpec((F, F), lambda i: (0, 0)),
            pl.BlockSpec((1, F), lambda i: (0, 0)),
        ],
        out_specs=[
            pl.BlockSpec((BR, FH), lambda i: (i, 0)),
            pl.BlockSpec((BR, FH), lambda i: (i, 0)),
        ],
        out_shape=[
            jax.ShapeDtypeStruct((2 * N, FH), jnp.float32),
            jax.ShapeDtypeStruct((2 * N, FH), jnp.float32),
        ],
    )(h, f_dst, wr, ur, br)


@functools.cache
def _make_segment_sums():
    mesh = plsc.VectorSubcoreMesh(core_axis_name="c", subcore_axis_name="s")

    @functools.partial(
        pl.kernel,
        out_type=[
            jax.ShapeDtypeStruct((2, OUTN, FH), jnp.float32),
            jax.ShapeDtypeStruct((2, OUTN, FH), jnp.float32),
        ],
        mesh=mesh,
        compiler_params=pltpu.CompilerParams(use_tc_tiling_on_sc=False),
        scratch_types=[
            pltpu.VMEM((NCHUNK, C), jnp.int32),
            pltpu.VMEM((NCHUNK, C), jnp.int32),
            pltpu.VMEM((C, FH), jnp.float32),
            pltpu.VMEM((C, FH), jnp.float32),
            pltpu.VMEM_SHARED((NACC, FH), jnp.float32),
            pltpu.SemaphoreType.DMA,
            pltpu.SemaphoreType.DMA,
        ],
    )
    def seg(ta_hbm, tb_hbm, src_hbm, dst_hbm, zeros_hbm, outa_hbm, outb_hbm,
            src_v, dst_v, rows_a, rows_b, acc_sh, sem_a, sem_b):
        c = lax.axis_index("c")
        s = lax.axis_index("s")
        pltpu.sync_copy(src_hbm.at[c, s], src_v)
        pltpu.sync_copy(dst_hbm.at[s], dst_v)

        for t_hbm, out_hbm in ((ta_hbm, outa_hbm), (tb_hbm, outb_hbm)):
            pltpu.sync_copy(zeros_hbm, acc_sh.at[pl.ds(s * ZROWS, ZROWS)])
            plsc.subcore_barrier()

            def body(p, carry):
                j = 2 * p
                da = pltpu.async_copy(t_hbm.at[src_v.at[j]], rows_a, sem_a)
                db = pltpu.async_copy(t_hbm.at[src_v.at[j + 1]], rows_b, sem_b)
                da.wait()
                pltpu.sync_copy(rows_a, acc_sh.at[dst_v.at[j]], add=True)
                db.wait()
                pltpu.sync_copy(rows_b, acc_sh.at[dst_v.at[j + 1]], add=True)
                return carry

            lax.fori_loop(0, NPAIR, body, 0)
            plsc.subcore_barrier()
            pltpu.sync_copy(acc_sh.at[pl.ds(s * ORS, ORS)],
                            out_hbm.at[c, pl.ds(s * ORS, ORS)])
            plsc.subcore_barrier()

    return seg


def _segment_sums(ta, tb, src2, dst_r, zeros):
    return _make_segment_sums()(ta, tb, src2, dst_r, zeros)


def _gate_body(fs_ref, sa_ref, sb_ref, ra_ref, rb_ref, wz_ref, uz_ref, bz_ref,
               w_ref, u_ref, b_ref, out_ref):
    fs = fs_ref[...]
    sv = jnp.concatenate([sa_ref[0], sb_ref[0]], axis=1)
    srh = jnp.concatenate([ra_ref[0], rb_ref[0]], axis=1)
    z = jax.nn.sigmoid(
        jnp.dot(fs, wz_ref[...], preferred_element_type=jnp.float32)
        + jnp.dot(sv, uz_ref[...], preferred_element_type=jnp.float32)
        + bz_ref[...]
    )
    ht = jnp.tanh(
        jnp.dot(fs, w_ref[...], preferred_element_type=jnp.float32)
        + jnp.dot(srh, u_ref[...], preferred_element_type=jnp.float32)
        + b_ref[...]
    )
    out_ref[...] = (1.0 - z) * sv + z * ht


def _gate(f_src, sega, segb, wz, uz, bz, w, u, b):
    full = lambda i: (0, 0)
    return pl.pallas_call(
        _gate_body,
        grid=(NB,),
        in_specs=[
            pl.BlockSpec((BR, F), lambda i: (i, 0)),
            pl.BlockSpec((1, BR, FH), lambda i: (0, i, 0)),
            pl.BlockSpec((1, BR, FH), lambda i: (0, i, 0)),
            pl.BlockSpec((1, BR, FH), lambda i: (1, i, 0)),
            pl.BlockSpec((1, BR, FH), lambda i: (1, i, 0)),
            pl.BlockSpec((F, F), full),
            pl.BlockSpec((F, F), full),
            pl.BlockSpec((1, F), full),
            pl.BlockSpec((F, F), full),
            pl.BlockSpec((F, F), full),
            pl.BlockSpec((1, F), full),
        ],
        out_specs=pl.BlockSpec((BR, F), lambda i: (i, 0)),
        out_shape=jax.ShapeDtypeStruct((N, F), jnp.float32),
    )(f_src, sega, segb, sega, segb, wz, uz, bz, w, u, b)


def kernel(h, f_src, f_dst, edge_index, wz, uz, bz, wr, ur, br, w, u, b):
    src = edge_index[0]
    dst = edge_index[1]
    pad = EPAD - E
    src_p = jnp.concatenate([src, jnp.zeros((pad,), jnp.int32)])
    dst_p = jnp.concatenate(
        [dst, N + (jnp.arange(pad, dtype=jnp.int32) % (OUTN - N))])
    src_r = src_p.reshape(TILES, NCHUNK, C)
    src2 = jnp.stack([src_r, src_r + N])
    dst_r = dst_p.reshape(TILES, NCHUNK, C)
    zeros = jnp.zeros((ZROWS, FH), jnp.float32)

    ta, tb = _build_tables(h, f_dst, wr, ur, br)
    sega, segb = _segment_sums(ta, tb, src2, dst_r, zeros)
    return _gate(f_src, sega, segb, wz, uz, bz, w, u, b)


# speedup vs baseline: 1.3821x; 1.3821x over previous
import functools

import jax
import jax.numpy as jnp
from jax import lax
from jax.experimental import pallas as pl
from jax.experimental.pallas import tpu as pltpu
from jax.experimental.pallas import tpu_sc as plsc

N = 10000
E = 320000
F = 128
FH = F // 2

C = 128
TILES = 16
NCHUNK = 157
EPAD = TILES * NCHUNK * C
OUTN = 10240
NACC = OUTN
ZROWS = NACC // TILES
ORS = OUTN // TILES

BR = 1000
NB = N // BR


def _build_tables_body(h_ref, fd_ref, wr_ref, ur_ref, br_ref, ta_ref, tb_ref):
    i = pl.program_id(0)

    @pl.when(i < NB)
    def _copy():
        hv = h_ref[...]
        ta_ref[...] = hv[:, :FH]
        tb_ref[...] = hv[:, FH:]

    @pl.when(i >= NB)
    def _compute():
        hv = h_ref[...]
        r = jax.nn.sigmoid(
            jnp.dot(fd_ref[...], wr_ref[...], preferred_element_type=jnp.float32)
            + jnp.dot(hv, ur_ref[...], preferred_element_type=jnp.float32)
            + br_ref[...]
        )
        rh = r * hv
        ta_ref[...] = rh[:, :FH]
        tb_ref[...] = rh[:, FH:]


def _build_tables(h, f_dst, wr, ur, br):
    return pl.pallas_call(
        _build_tables_body,
        grid=(2 * NB,),
        in_specs=[
            pl.BlockSpec((BR, F), lambda i: (jnp.where(i < NB, i, i - NB), 0)),
            pl.BlockSpec((BR, F), lambda i: (jnp.where(i < NB, 0, i - NB), 0)),
            pl.BlockSpec((F, F), lambda i: (0, 0)),
            pl.BlockSpec((F, F), lambda i: (0, 0)),
            pl.BlockSpec((1, F), lambda i: (0, 0)),
        ],
        out_specs=[
            pl.BlockSpec((BR, FH), lambda i: (i, 0)),
            pl.BlockSpec((BR, FH), lambda i: (i, 0)),
        ],
        out_shape=[
            jax.ShapeDtypeStruct((2 * N, FH), jnp.float32),
            jax.ShapeDtypeStruct((2 * N, FH), jnp.float32),
        ],
    )(h, f_dst, wr, ur, br)


@functools.cache
def _make_segment_sums():
    mesh = plsc.VectorSubcoreMesh(core_axis_name="c", subcore_axis_name="s")

    @functools.partial(
        pl.kernel,
        out_type=[
            jax.ShapeDtypeStruct((2, OUTN, FH), jnp.float32),
            jax.ShapeDtypeStruct((2, OUTN, FH), jnp.float32),
        ],
        mesh=mesh,
        compiler_params=pltpu.CompilerParams(use_tc_tiling_on_sc=False),
        scratch_types=[
            pltpu.VMEM((NCHUNK, C), jnp.int32),
            pltpu.VMEM((NCHUNK, C), jnp.int32),
            pltpu.VMEM((C, FH), jnp.float32),
            pltpu.VMEM_SHARED((NACC, FH), jnp.float32),
            pltpu.SemaphoreType.DMA,
        ],
    )
    def seg(ta_hbm, tb_hbm, src_hbm, dst_hbm, zeros_hbm, outa_hbm, outb_hbm,
            src_v, dst_v, rows_v, acc_sh, sem):
        c = lax.axis_index("c")
        s = lax.axis_index("s")
        pltpu.sync_copy(src_hbm.at[c, s], src_v)
        pltpu.sync_copy(dst_hbm.at[s], dst_v)

        for t_hbm, out_hbm in ((ta_hbm, outa_hbm), (tb_hbm, outb_hbm)):
            pltpu.sync_copy(zeros_hbm, acc_sh.at[pl.ds(s * ZROWS, ZROWS)])
            plsc.subcore_barrier()

            def body(j, carry):
                pltpu.async_copy(t_hbm.at[src_v.at[j]], rows_v, sem).wait()
                pltpu.sync_copy(rows_v, acc_sh.at[dst_v.at[j]], add=True)
                return carry

            lax.fori_loop(0, NCHUNK, body, 0)
            plsc.subcore_barrier()
            pltpu.sync_copy(acc_sh.at[pl.ds(s * ORS, ORS)],
                            out_hbm.at[c, pl.ds(s * ORS, ORS)])
            plsc.subcore_barrier()

    return seg


def _segment_sums(ta, tb, src2, dst_r, zeros):
    return _make_segment_sums()(ta, tb, src2, dst_r, zeros)


def _gate_body(fs_ref, sa_ref, sb_ref, ra_ref, rb_ref, wz_ref, uz_ref, bz_ref,
               w_ref, u_ref, b_ref, out_ref):
    fs = fs_ref[...]
    sv = jnp.concatenate([sa_ref[0], sb_ref[0]], axis=1)
    srh = jnp.concatenate([ra_ref[0], rb_ref[0]], axis=1)
    z = jax.nn.sigmoid(
        jnp.dot(fs, wz_ref[...], preferred_element_type=jnp.float32)
        + jnp.dot(sv, uz_ref[...], preferred_element_type=jnp.float32)
        + bz_ref[...]
    )
    ht = jnp.tanh(
        jnp.dot(fs, w_ref[...], preferred_element_type=jnp.float32)
        + jnp.dot(srh, u_ref[...], preferred_element_type=jnp.float32)
        + b_ref[...]
    )
    out_ref[...] = (1.0 - z) * sv + z * ht


def _gate(f_src, sega, segb, wz, uz, bz, w, u, b):
    full = lambda i: (0, 0)
    return pl.pallas_call(
        _gate_body,
        grid=(NB,),
        in_specs=[
            pl.BlockSpec((BR, F), lambda i: (i, 0)),
            pl.BlockSpec((1, BR, FH), lambda i: (0, i, 0)),
            pl.BlockSpec((1, BR, FH), lambda i: (0, i, 0)),
            pl.BlockSpec((1, BR, FH), lambda i: (1, i, 0)),
            pl.BlockSpec((1, BR, FH), lambda i: (1, i, 0)),
            pl.BlockSpec((F, F), full),
            pl.BlockSpec((F, F), full),
            pl.BlockSpec((1, F), full),
            pl.BlockSpec((F, F), full),
            pl.BlockSpec((F, F), full),
            pl.BlockSpec((1, F), full),
        ],
        out_specs=pl.BlockSpec((BR, F), lambda i: (i, 0)),
        out_shape=jax.ShapeDtypeStruct((N, F), jnp.float32),
    )(f_src, sega, segb, sega, segb, wz, uz, bz, w, u, b)


def kernel(h, f_src, f_dst, edge_index, wz, uz, bz, wr, ur, br, w, u, b):
    src = edge_index[0]
    dst = edge_index[1]
    pad = EPAD - E
    src_p = jnp.concatenate([src, jnp.zeros((pad,), jnp.int32)])
    dst_p = jnp.concatenate(
        [dst, N + (jnp.arange(pad, dtype=jnp.int32) % (OUTN - N))])
    src_r = src_p.reshape(TILES, NCHUNK, C)
    src2 = jnp.stack([src_r, src_r + N])
    dst_r = dst_p.reshape(TILES, NCHUNK, C)
    zeros = jnp.zeros((ZROWS, FH), jnp.float32)

    ta, tb = _build_tables(h, f_dst, wr, ur, br)
    sega, segb = _segment_sums(ta, tb, src2, dst_r, zeros)
    return _gate(f_src, sega, segb, wz, uz, bz, w, u, b)
